# baseline (speedup 1.0000x reference)
import functools

import jax
import jax.numpy as jnp
from jax import lax
from jax.experimental import pallas as pl
from jax.experimental.pallas import tpu as pltpu
from jax.experimental.pallas import tpu_sc as plsc

_B, _L, _V, _D = 16, 4096, 1000000, 128
_R = 2 * _B
_NT = _R * _L
_NC, _NS = 2, 16
_NW = _NC * _NS
_TPW = _NT // _NW
_CH = 128

def _wid():
    return lax.axis_index("s") * _NC + lax.axis_index("c")


_NCH = _TPW // _CH
_ACC = _CH // 16


@functools.cache
def _make_sc_gather_dot():
    mesh = plsc.VectorSubcoreMesh(core_axis_name="c", subcore_axis_name="s")

    @functools.partial(
        pl.kernel,
        mesh=mesh,
        compiler_params=pltpu.CompilerParams(needs_layout_passes=False),
        out_type=jax.ShapeDtypeStruct((_R, _L), jnp.float32),
        scratch_types=[
            pltpu.VMEM((_TPW,), jnp.int32),
            pltpu.VMEM((_CH, _D), jnp.float32),
            pltpu.VMEM((_CH, _D), jnp.float32),
            pltpu.VMEM((_D, 16), jnp.float32),
            pltpu.VMEM((_TPW,), jnp.float32),
            pltpu.SemaphoreType.DMA,
            pltpu.SemaphoreType.DMA,
        ],
    )
    def _sc_gather_dot(tok_hbm, wexp_hbm, emb_hbm, out_hbm,
                       idx_v, rows0, rows1, w_v, sc_v, sem0, sem1):
        w = _wid()
        base = w * _TPW
        pltpu.sync_copy(tok_hbm.at[pl.ds(base, _TPW)], idx_v)
        pltpu.sync_copy(wexp_hbm, w_v)

        def start(c, rows, sem):
            pltpu.async_copy(emb_hbm.at[idx_v.at[pl.ds(c * _CH, _CH)]],
                             rows, sem)

        def wait(rows, sem):
            pltpu.make_async_copy(emb_hbm.at[idx_v.at[pl.ds(0, _CH)]],
                                  rows, sem).wait()

        row_ids = [lax.iota(jnp.int32, 16) + a * 16 for a in range(_ACC)]

        def compute(c, rows):
            def jbody(j, accs):
                wj = w_v[j]
                colj = jnp.full((16,), j, jnp.int32)
                return tuple(
                    accs[a] + plsc.load_gather(rows, [row_ids[a], colj]) * wj
                    for a in range(_ACC)
                )

            accs = lax.fori_loop(
                0, _D, jbody,
                tuple(jnp.zeros((16,), jnp.float32) for _ in range(_ACC)),
            )
            for a in range(_ACC):
                sc_v[pl.ds(c * _CH + a * 16, 16)] = accs[a]

        start(0, rows0, sem0)

        def pair(i, carry):
            c0 = 2 * i
            start(c0 + 1, rows1, sem1)
            wait(rows0, sem0)
            compute(c0, rows0)
            start(c0 + 2, rows0, sem0)
            wait(rows1, sem1)
            compute(c0 + 1, rows1)
            return carry

        lax.fori_loop(0, _NCH // 2 - 1, pair, 0)
        start(_NCH - 1, rows1, sem1)
        wait(rows0, sem0)
        compute(_NCH - 2, rows0)
        wait(rows1, sem1)
        compute(_NCH - 1, rows1)
        pltpu.sync_copy(sc_v, out_hbm.at[w])

    return _sc_gather_dot


def _post_body(tok_ref, sc_ref, u_ref, b_ref, sel_ref, len_ref, logp_ref, zs_ref, zd_ref):
    eps = 1e-8
    p = jax.nn.sigmoid(sc_ref[...] + b_ref[0, 0])
    sel = (u_ref[...] < p).astype(jnp.int32)
    sel_ref[...] = sel
    len_ref[...] = jnp.sum(sel, axis=1, keepdims=True)
    nz = tok_ref[...] != 0
    mf = nz.astype(jnp.float32)
    self_f = sel.astype(jnp.float32)
    logp = self_f * jnp.log(p + eps) + (1.0 - self_f) * jnp.log(1.0 - p + eps)
    logp_ref[...] = jnp.sum(logp * mf, axis=1, keepdims=True)
    ms = sel * nz.astype(jnp.int32)
    zs_ref[...] = jnp.sum(ms, axis=1, keepdims=True).astype(jnp.float32)
    d = jnp.abs(ms[:, 1:] - ms[:, :-1])
    zd_ref[...] = jnp.sum(d, axis=1, keepdims=True).astype(jnp.float32)


_tc_post = pl.pallas_call(
    _post_body,
    out_shape=(
        jax.ShapeDtypeStruct((_R, _L), jnp.int32),
        jax.ShapeDtypeStruct((_R, 1), jnp.int32),
        jax.ShapeDtypeStruct((_R, 1), jnp.float32),
        jax.ShapeDtypeStruct((_R, 1), jnp.float32),
        jax.ShapeDtypeStruct((_R, 1), jnp.float32),
    ),
)


@functools.cache
def _make_sc_compact():
    mesh = plsc.VectorSubcoreMesh(core_axis_name="c", subcore_axis_name="s")

    @functools.partial(
        pl.kernel,
        mesh=mesh,
        compiler_params=pltpu.CompilerParams(needs_layout_passes=False),
        out_type=jax.ShapeDtypeStruct((_R, _L), jnp.int32),
        scratch_types=[
            pltpu.VMEM((_L,), jnp.int32),
            pltpu.VMEM((_L,), jnp.int32),
            pltpu.VMEM((_L,), jnp.int32),
        ],
    )
    def _sc_compact(tok_hbm, sel_hbm, out_hbm, tok_v, sel_v, out_v):
        row = _wid()
        pltpu.sync_copy(tok_hbm.at[row], tok_v)
        pltpu.sync_copy(sel_hbm.at[row], sel_v)

        zero = jnp.zeros((16,), jnp.int32)

        def zbody(i, carry):
            out_v[pl.ds(i * 16, 16)] = zero
            return carry

        lax.fori_loop(0, _L // 16, zbody, 0)

        def body(i, off):
            s = sel_v[pl.ds(i * 16, 16)]
            t = tok_v[pl.ds(i * 16, 16)]
            pos = plsc.cumsum(s) - 1 + off
            plsc.store_scatter(out_v, [pos], t, mask=s > 0)
            return off + jnp.sum(s)

        lax.fori_loop(0, _L // 16, body, jnp.int32(0))
        pltpu.sync_copy(out_v, out_hbm.at[row])

    return _sc_compact


def kernel(sentence1, sentence1_len_old, sentence2, sentence2_len_old,
           emb_table, W_sel, b_sel, is_train):
    key = jax.random.key(42)
    k1, k2 = jax.random.split(key)
    u1 = jax.random.uniform(k1, (_B, _L))
    u2 = jax.random.uniform(k2, (_B, _L))
    u = jnp.concatenate([u1, u2], axis=0)
    tok = jnp.concatenate([sentence1, sentence2], axis=0)

    wexp = jnp.broadcast_to(W_sel, (_D, 16))
    scores = _make_sc_gather_dot()(tok.reshape(_NT), wexp, emb_table)
    sel, lens, logp, zs, zd = _tc_post(tok, scores, u, b_sel.reshape(1, 1))
    selected = _make_sc_compact()(tok, sel)

    logpz = logp[:_B, 0] + logp[_B:, 0]
    zsum = zs[:_B, 0] + zs[_B:, 0]
    zdiff = zd[:_B, 0] + zd[_B:, 0]
    flag = is_train == 1
    logpz = jnp.where(flag, logpz, -1.0)
    zsum = jnp.where(flag, zsum, -1.0)
    zdiff = jnp.where(flag, zdiff, -1.0)
    return (selected[:_B], lens[:_B, 0], selected[_B:], lens[_B:, 0],
            logpz, zsum, zdiff)
